# baseline (speedup 1.0000x reference)
import functools

import jax
import jax.numpy as jnp
from jax import lax
from jax.experimental import pallas as pl
from jax.experimental.pallas import tpu as pltpu
from jax.experimental.pallas import tpu_sc as plsc

NTOKEN = 100000
D = 128
NC = 2
NS = 16
NW = NC * NS
CHUNK = 200
N_BUF = 4
AHEAD = 3


def _make_gather(B: int):
  b_per_w = B // NW
  n_steps = b_per_w // CHUNK
  mesh = plsc.VectorSubcoreMesh(
      core_axis_name="c", subcore_axis_name="s", num_cores=NC, num_subcores=NS
  )

  @functools.partial(
      pl.kernel,
      mesh=mesh,
      out_type=jax.ShapeDtypeStruct((B, D), jnp.float32),
      scratch_types=[
          pltpu.VMEM((b_per_w,), jnp.int32),
          [pltpu.VMEM((CHUNK, D), jnp.float32) for _ in range(N_BUF)],
          [pltpu.SemaphoreType.DMA for _ in range(N_BUF)],
          [pltpu.SemaphoreType.DMA for _ in range(N_BUF)],
      ],
  )
  def gather_kernel(idx_hbm, table_hbm, out_hbm, idx_v, bufs, gsems, wsems):
    wid = lax.axis_index("s") * NC + lax.axis_index("c")
    base = wid * b_per_w
    pltpu.sync_copy(idx_hbm.at[pl.ds(base, b_per_w)], idx_v)

    def start_gather(step):
      b = step % N_BUF
      pltpu.async_copy(
          table_hbm.at[idx_v.at[pl.ds(step * CHUNK, CHUNK)]], bufs[b], gsems[b]
      )

    def out_slice(step):
      return out_hbm.at[pl.ds(base + step * CHUNK, CHUNK)]

    unwaited_write = [False] * N_BUF
    for j in range(min(AHEAD, n_steps)):
      start_gather(j)
    for i in range(n_steps):
      b = i % N_BUF
      nxt = i + AHEAD
      if nxt < n_steps:
        bn = nxt % N_BUF
        if nxt >= N_BUF:
          pltpu.make_async_copy(bufs[bn], out_slice(nxt - N_BUF), wsems[bn]).wait()
          unwaited_write[bn] = False
        start_gather(nxt)
      pltpu.make_async_copy(
          table_hbm.at[idx_v.at[pl.ds(i * CHUNK, CHUNK)]], bufs[b], gsems[b]
      ).wait()
      pltpu.async_copy(bufs[b], out_slice(i), wsems[b])
      unwaited_write[b] = True
    for b in range(N_BUF):
      if unwaited_write[b]:
        pltpu.make_async_copy(bufs[b], out_slice(0), wsems[b]).wait()

  return gather_kernel


@jax.jit
def kernel(input, table):
  nseq, seq = input.shape
  flat_idx = input.T.reshape(-1).astype(jnp.int32)
  out = _make_gather(flat_idx.shape[0])(flat_idx, table)
  return out.reshape(seq, nseq, D).transpose(1, 0, 2)


# speedup vs baseline: 1.0078x; 1.0078x over previous
import functools

import jax
import jax.numpy as jnp
from jax import lax
from jax.experimental import pallas as pl
from jax.experimental.pallas import tpu as pltpu
from jax.experimental.pallas import tpu_sc as plsc

NTOKEN = 100000
D = 128
NC = 2
NS = 16
NW = NC * NS
CHUNK = 320
N_BUF = 3
AHEAD = 2
assert AHEAD <= N_BUF - 1


def _make_gather(B: int):
  b_per_w = B // NW
  n_steps = b_per_w // CHUNK
  mesh = plsc.VectorSubcoreMesh(
      core_axis_name="c", subcore_axis_name="s", num_cores=NC, num_subcores=NS
  )

  @functools.partial(
      pl.kernel,
      mesh=mesh,
      out_type=jax.ShapeDtypeStruct((B, D), jnp.float32),
      scratch_types=[
          pltpu.VMEM((b_per_w,), jnp.int32),
          [pltpu.VMEM((CHUNK, D), jnp.float32) for _ in range(N_BUF)],
          [pltpu.SemaphoreType.DMA for _ in range(N_BUF)],
          [pltpu.SemaphoreType.DMA for _ in range(N_BUF)],
      ],
  )
  def gather_kernel(idx_hbm, table_hbm, out_hbm, idx_v, bufs, gsems, wsems):
    wid = lax.axis_index("s") * NC + lax.axis_index("c")
    base = wid * b_per_w
    pltpu.sync_copy(idx_hbm.at[pl.ds(base, b_per_w)], idx_v)

    def start_gather(step):
      b = step % N_BUF
      pltpu.async_copy(
          table_hbm.at[idx_v.at[pl.ds(step * CHUNK, CHUNK)]], bufs[b], gsems[b]
      )

    def out_slice(step):
      return out_hbm.at[pl.ds(base + step * CHUNK, CHUNK)]

    unwaited_write = [False] * N_BUF
    for j in range(min(AHEAD, n_steps)):
      start_gather(j)
    for i in range(n_steps):
      b = i % N_BUF
      nxt = i + AHEAD
      if nxt < n_steps:
        bn = nxt % N_BUF
        if nxt >= N_BUF:
          pltpu.make_async_copy(bufs[bn], out_slice(nxt - N_BUF), wsems[bn]).wait()
          unwaited_write[bn] = False
        start_gather(nxt)
      pltpu.make_async_copy(
          table_hbm.at[idx_v.at[pl.ds(i * CHUNK, CHUNK)]], bufs[b], gsems[b]
      ).wait()
      pltpu.async_copy(bufs[b], out_slice(i), wsems[b])
      unwaited_write[b] = True
    for b in range(N_BUF):
      if unwaited_write[b]:
        pltpu.make_async_copy(bufs[b], out_slice(0), wsems[b]).wait()

  return gather_kernel


@jax.jit
def kernel(input, table):
  nseq, seq = input.shape
  flat_idx = input.T.reshape(-1).astype(jnp.int32)
  out = _make_gather(flat_idx.shape[0])(flat_idx, table)
  return out.reshape(seq, nseq, D).transpose(1, 0, 2)
